# baseline (speedup 1.0000x reference)
import functools

import jax
import jax.numpy as jnp
from jax import lax
from jax.experimental import pallas as pl
from jax.experimental.pallas import tpu as pltpu
from jax.experimental.pallas import tpu_sc as plsc

N = 32768
D = 768
E = 8
NW = 32
CHUNK = 64
GROUPS = CHUNK // 16

N_SC = 16384
N_TC = N - N_SC
TC_BLK = 1024


def _make_sc_call(n_rows):
    ROWS_W = n_rows // NW
    NCHUNK = ROWS_W // CHUNK
    mesh = plsc.VectorSubcoreMesh(core_axis_name="c", subcore_axis_name="s")

    @functools.partial(
        pl.kernel,
        out_type=jax.ShapeDtypeStruct((n_rows,), jnp.float32),
        mesh=mesh,
        scratch_types=[
            pltpu.VMEM((E, D), jnp.float32),
            pltpu.VMEM((16,), jnp.float32),
            pltpu.VMEM((ROWS_W,), jnp.int32),
            pltpu.VMEM((ROWS_W,), jnp.float32),
            pltpu.VMEM((CHUNK, D), jnp.float32),
            pltpu.VMEM((CHUNK, D), jnp.float32),
            pltpu.SemaphoreType.DMA,
            pltpu.SemaphoreType.DMA,
        ],
        compiler_params=pltpu.CompilerParams(needs_layout_passes=False),
    )
    def run(element_hbm, desc_hbm, w_hbm, b_hbm, out_hbm,
            w_v, b_v, e_v, o_v, dbuf0, dbuf1, sem0, sem1):
        nc = 2
        wid = lax.axis_index("s") * nc + lax.axis_index("c")
        base = wid * ROWS_W

        pltpu.sync_copy(w_hbm, w_v)
        pltpu.sync_copy(b_hbm, b_v)
        pltpu.sync_copy(element_hbm.at[pl.ds(base, ROWS_W)], e_v)

        lane = lax.iota(jnp.int32, 16)
        zero16 = jnp.zeros((16,), jnp.float32)

        def do_group(dbuf, row0, buf_row0):
            e_vec = e_v[pl.ds(row0, 16)]
            bias_vec = plsc.load_gather(b_v, [e_vec])
            erow = [jnp.max(jnp.where(lane == r, e_vec, 0))
                    for r in range(16)]

            def blk(i, accs):
                accs = list(accs)
                off = i * 16
                for r in range(16):
                    d = dbuf[buf_row0 + r, pl.ds(off, 16)]
                    w = w_v[erow[r], pl.ds(off, 16)]
                    accs[r] = accs[r] + d * w
                return tuple(accs)

            accs = lax.fori_loop(0, D // 16, blk, (zero16,) * 16)
            out_vec = bias_vec
            for r in range(16):
                s = jnp.sum(accs[r])
                out_vec = jnp.where(lane == r, bias_vec + s, out_vec)
            o_v[pl.ds(row0, 16)] = out_vec

        pltpu.async_copy(
            desc_hbm.at[pl.ds(base, CHUNK), :], dbuf0, sem0)
        pltpu.async_copy(
            desc_hbm.at[pl.ds(base + CHUNK, CHUNK), :], dbuf1, sem1)

        def chunk_pair(i, carry):
            for half, (buf, sem) in enumerate(((dbuf0, sem0), (dbuf1, sem1))):
                k = 2 * i + half
                pltpu.make_async_copy(
                    desc_hbm.at[pl.ds(0, CHUNK), :], buf, sem).wait()

                def groups(g, c, buf=buf, k=k):
                    do_group(buf, k * CHUNK + g * 16, g * 16)
                    return c

                lax.fori_loop(0, GROUPS, groups, 0)

                nk = k + 2

                @pl.when(nk < NCHUNK)
                def _(buf=buf, sem=sem, nk=nk):
                    pltpu.async_copy(
                        desc_hbm.at[pl.ds(base + nk * CHUNK, CHUNK), :],
                        buf, sem)
            return carry

        lax.fori_loop(0, NCHUNK // 2, chunk_pair, 0)

        pltpu.sync_copy(o_v, out_hbm.at[pl.ds(base, ROWS_W)])

    return run


_sc_call = _make_sc_call(N_SC)


def _tc_body(e_ref, d_ref, w_ref, b_ref, o_ref):
    mat = jax.lax.dot_general(
        d_ref[...], w_ref[...], (((1,), (1,)), ((), ())),
        preferred_element_type=jnp.float32)
    e = e_ref[...]
    cols = lax.broadcasted_iota(jnp.int32, (TC_BLK, E), 1)
    sel = cols == e[:, None]
    o_ref[...] = jnp.sum(
        jnp.where(sel, mat + b_ref[...], 0.0), axis=1)


_TC_OFF = N_SC // TC_BLK

_tc_call = pl.pallas_call(
    _tc_body,
    grid=(N_TC // TC_BLK,),
    in_specs=[
        pl.BlockSpec((TC_BLK,), lambda i: (i + _TC_OFF,)),
        pl.BlockSpec((TC_BLK, D), lambda i: (i + _TC_OFF, 0)),
        pl.BlockSpec((E, D), lambda i: (0, 0)),
        pl.BlockSpec((1, E), lambda i: (0, 0)),
    ],
    out_specs=pl.BlockSpec((TC_BLK,), lambda i: (i,)),
    out_shape=jax.ShapeDtypeStruct((N_TC,), jnp.float32),
)


def kernel(element, desc, W, b):
    b_pad = jnp.pad(b, (0, 16 - E))
    out_sc = _sc_call(element, desc, W, b_pad)
    out_tc = _tc_call(element, desc, W, b.reshape(1, E))
    return jnp.concatenate([out_sc, out_tc])


# speedup vs baseline: 1.0738x; 1.0738x over previous
import functools

import jax
import jax.numpy as jnp
from jax import lax
from jax.experimental import pallas as pl
from jax.experimental.pallas import tpu as pltpu
from jax.experimental.pallas import tpu_sc as plsc

N = 32768
D = 768
E = 8
NW = 32
CHUNK = 48
GROUPS = CHUNK // 16

N_SC = 15360
N_TC = N - N_SC
TC_BLK = 1024


def _make_sc_call(n_rows):
    ROWS_W = n_rows // NW
    NCHUNK = ROWS_W // CHUNK
    mesh = plsc.VectorSubcoreMesh(core_axis_name="c", subcore_axis_name="s")

    @functools.partial(
        pl.kernel,
        out_type=jax.ShapeDtypeStruct((n_rows,), jnp.float32),
        mesh=mesh,
        scratch_types=[
            pltpu.VMEM((E, D), jnp.float32),
            pltpu.VMEM((16,), jnp.float32),
            pltpu.VMEM((ROWS_W,), jnp.int32),
            pltpu.VMEM((ROWS_W,), jnp.float32),
            pltpu.VMEM((CHUNK, D), jnp.float32),
            pltpu.VMEM((CHUNK, D), jnp.float32),
            pltpu.SemaphoreType.DMA,
            pltpu.SemaphoreType.DMA,
        ],
        compiler_params=pltpu.CompilerParams(needs_layout_passes=False),
    )
    def run(element_hbm, desc_hbm, w_hbm, b_hbm, out_hbm,
            w_v, b_v, e_v, o_v, dbuf0, dbuf1, sem0, sem1):
        nc = 2
        wid = lax.axis_index("s") * nc + lax.axis_index("c")
        base = wid * ROWS_W

        pltpu.sync_copy(w_hbm, w_v)
        pltpu.sync_copy(b_hbm, b_v.at[pl.ds(0, E)])
        pltpu.sync_copy(element_hbm.at[pl.ds(base, ROWS_W)], e_v)

        lane = lax.iota(jnp.int32, 16)
        zero16 = jnp.zeros((16,), jnp.float32)

        def do_group(dbuf, row0, buf_row0):
            e_vec = e_v[pl.ds(row0, 16)]
            bias_vec = plsc.load_gather(b_v, [e_vec])
            erow = [jnp.max(jnp.where(lane == r, e_vec, 0))
                    for r in range(16)]

            def blk(i, accs):
                accs = list(accs)
                off = i * 16
                for r in range(16):
                    d = dbuf[buf_row0 + r, pl.ds(off, 16)]
                    w = w_v[erow[r], pl.ds(off, 16)]
                    accs[r] = accs[r] + d * w
                return tuple(accs)

            accs = lax.fori_loop(0, D // 16, blk, (zero16,) * 16)
            out_vec = bias_vec
            for r in range(16):
                s = jnp.sum(accs[r])
                out_vec = jnp.where(lane == r, bias_vec + s, out_vec)
            o_v[pl.ds(row0, 16)] = out_vec

        pltpu.async_copy(
            desc_hbm.at[pl.ds(base, CHUNK), :], dbuf0, sem0)
        pltpu.async_copy(
            desc_hbm.at[pl.ds(base + CHUNK, CHUNK), :], dbuf1, sem1)

        def chunk_pair(i, carry):
            for half, (buf, sem) in enumerate(((dbuf0, sem0), (dbuf1, sem1))):
                k = 2 * i + half
                pltpu.make_async_copy(
                    desc_hbm.at[pl.ds(0, CHUNK), :], buf, sem).wait()

                def groups(g, c, buf=buf, k=k):
                    do_group(buf, k * CHUNK + g * 16, g * 16)
                    return c

                lax.fori_loop(0, GROUPS, groups, 0)

                nk = k + 2

                @pl.when(nk < NCHUNK)
                def _(buf=buf, sem=sem, nk=nk):
                    pltpu.async_copy(
                        desc_hbm.at[pl.ds(base + nk * CHUNK, CHUNK), :],
                        buf, sem)
            return carry

        lax.fori_loop(0, NCHUNK // 2, chunk_pair, 0)

        pltpu.sync_copy(o_v, out_hbm.at[pl.ds(base, ROWS_W)])

    return run


_sc_call = _make_sc_call(N_SC)


def _tc_body(e_ref, d_ref, w_ref, b_ref, o_ref):
    mat = jax.lax.dot_general(
        d_ref[...], w_ref[...], (((1,), (1,)), ((), ())),
        preferred_element_type=jnp.float32)
    e = e_ref[...]
    cols = lax.broadcasted_iota(jnp.int32, (TC_BLK, E), 1)
    sel = cols == e[:, None]
    o_ref[...] = jnp.sum(
        jnp.where(sel, mat + b_ref[...], 0.0), axis=1)


_TC_OFF = N_SC // TC_BLK

_tc_call = pl.pallas_call(
    _tc_body,
    grid=(N_TC // TC_BLK,),
    in_specs=[
        pl.BlockSpec((TC_BLK,), lambda i: (i + _TC_OFF,)),
        pl.BlockSpec((TC_BLK, D), lambda i: (i + _TC_OFF, 0)),
        pl.BlockSpec((E, D), lambda i: (0, 0)),
        pl.BlockSpec((1, E), lambda i: (0, 0)),
    ],
    out_specs=pl.BlockSpec((TC_BLK,), lambda i: (i,)),
    out_shape=jax.ShapeDtypeStruct((N_TC,), jnp.float32),
)


def kernel(element, desc, W, b):
    out_sc = _sc_call(element, desc, W, b)
    out_tc = _tc_call(element, desc, W, b.reshape(1, E))
    return jnp.concatenate([out_sc, out_tc])
